# baseline (speedup 1.0000x reference)
import jax
import jax.numpy as jnp
from jax import lax
from jax.experimental import pallas as pl
from jax.experimental.pallas import tpu as pltpu
from jax.experimental.pallas import tpu_sc as plsc

N = 10000
E = 320000
D = 128

NC = 2
NS = 16
NW = NC * NS
EPW = E // NW
CH = 128
NCHUNK = 79
EPWP = NCHUNK * CH
PAD = EPWP - EPW
EP = NW * EPWP
NPAD = 10240
RPS = NPAD // NS


def _gather_body(x_hbm, src_hbm, dst_hbm, xs_hbm, xd_hbm,
                 idx_s, idx_d, bs, bd, g_sem, w_sem):
    wid = lax.axis_index("s") * NC + lax.axis_index("c")
    pltpu.sync_copy(src_hbm.at[wid], idx_s)
    pltpu.sync_copy(dst_hbm.at[wid], idx_d)

    def gather(j, p):
        return (pltpu.make_async_copy(x_hbm.at[idx_s.at[j]], bs.at[p],
                                      g_sem.at[p]),
                pltpu.make_async_copy(x_hbm.at[idx_d.at[j]], bd.at[p],
                                      g_sem.at[p]))

    def write(j, p):
        base = wid * EPWP + j * CH
        return (pltpu.make_async_copy(bs.at[p], xs_hbm.at[pl.ds(base, CH)],
                                      w_sem.at[p]),
                pltpu.make_async_copy(bd.at[p], xd_hbm.at[pl.ds(base, CH)],
                                      w_sem.at[p]))

    def step(j, carry):
        for d in gather(j, 0):
            d.start()
        for d in gather(j, 0):
            d.wait()
        for d in write(j, 0):
            d.start()
        for d in write(j, 0):
            d.wait()
        return carry

    lax.fori_loop(0, NCHUNK, step, 0)


_gather = pl.kernel(
    _gather_body,
    out_type=(jax.ShapeDtypeStruct((EP, D), jnp.float32),
              jax.ShapeDtypeStruct((EP, D), jnp.float32)),
    mesh=plsc.VectorSubcoreMesh(core_axis_name="c", subcore_axis_name="s"),
    scratch_types=[
        pltpu.VMEM((NCHUNK, CH), jnp.int32),
        pltpu.VMEM((NCHUNK, CH), jnp.int32),
        pltpu.VMEM((2, CH, D), jnp.float32),
        pltpu.VMEM((2, CH, D), jnp.float32),
        pltpu.SemaphoreType.DMA((2,)),
        pltpu.SemaphoreType.DMA((2,)),
    ],
)


def _mlp_body(xs_ref, xd_ref, w1_ref, b1_ref, w2r_ref, b2_ref, w4_ref,
              b4_ref, out_ref):
    xs = xs_ref[...]
    xd = xd_ref[...]
    theta = xd - xs
    h1 = jnp.dot(theta, w1_ref[...], preferred_element_type=jnp.float32)
    h1 = jnp.maximum(h1 + b1_ref[...], 0.0)
    logit = jnp.sum(h1 * w2r_ref[...], axis=1, keepdims=True) + b2_ref[0, 0]
    w = jax.nn.sigmoid(logit)
    xlin = jnp.dot(xd, w4_ref[...], preferred_element_type=jnp.float32)
    out_ref[...] = w * theta + xlin + b4_ref[...]


BE = 2048


def _mlp(xs, xd, W1, b1r, W2r, b2r, W4, b4r):
    full = lambda shape: pl.BlockSpec(shape, lambda i: (0, 0))
    return pl.pallas_call(
        _mlp_body,
        grid=(EP // BE,),
        in_specs=[
            pl.BlockSpec((BE, D), lambda i: (i, 0)),
            pl.BlockSpec((BE, D), lambda i: (i, 0)),
            full((D, D)),
            full((1, D)),
            full((1, D)),
            pl.BlockSpec(memory_space=pltpu.SMEM),
            full((D, D)),
            full((1, D)),
        ],
        out_specs=pl.BlockSpec((BE, D), lambda i: (i, 0)),
        out_shape=jax.ShapeDtypeStruct((EP, D), jnp.float32),
    )(xs, xd, W1, b1r, W2r, b2r, W4, b4r)


def _scatter_body(m_hbm, dst_hbm, z_hbm, out_hbm, idx_v, buf, acc, l_sem):
    c = lax.axis_index("c")
    s = lax.axis_index("s")
    wid = s * NC + c
    pltpu.sync_copy(z_hbm.at[pl.ds(s * RPS, RPS)], acc.at[pl.ds(s * RPS, RPS)])
    pltpu.sync_copy(dst_hbm.at[wid], idx_v)
    plsc.subcore_barrier()

    def load(j, p):
        base = wid * EPWP + j * CH
        return pltpu.make_async_copy(m_hbm.at[pl.ds(base, CH)], buf.at[p],
                                     l_sem.at[p])

    load(0, 0).start()

    def step(j, carry):
        p = lax.rem(j, 2)

        @pl.when(j + 1 < NCHUNK)
        def _():
            load(j + 1, 1 - p).start()

        load(j, p).wait()
        pltpu.sync_copy(buf.at[p], acc.at[idx_v.at[j]], add=True)
        return carry

    lax.fori_loop(0, NCHUNK, step, 0)
    plsc.subcore_barrier()
    pltpu.sync_copy(acc.at[pl.ds(s * RPS, RPS)],
                    out_hbm.at[c, pl.ds(s * RPS, RPS)])


_scatter = pl.kernel(
    _scatter_body,
    out_type=jax.ShapeDtypeStruct((NC, NPAD, D), jnp.float32),
    mesh=plsc.VectorSubcoreMesh(core_axis_name="c", subcore_axis_name="s"),
    scratch_types=[
        pltpu.VMEM((NCHUNK, CH), jnp.int32),
        pltpu.VMEM((2, CH, D), jnp.float32),
        pltpu.VMEM_SHARED((NPAD, D), jnp.float32),
        pltpu.SemaphoreType.DMA((2,)),
    ],
)


def _combine_body(p_ref, out_ref):
    out_ref[...] = p_ref[0] + p_ref[1]


BN = 1000


def _combine(partials):
    return pl.pallas_call(
        _combine_body,
        grid=(N // BN,),
        in_specs=[pl.BlockSpec((NC, BN, D), lambda i: (0, i, 0))],
        out_specs=pl.BlockSpec((BN, D), lambda i: (i, 0)),
        out_shape=jax.ShapeDtypeStruct((N, D), jnp.float32),
    )(partials)


def kernel(x, edge_index, W1, b1, W2, b2, W4, b4):
    src = edge_index[0].reshape(NW, EPW)
    dst = edge_index[1].reshape(NW, EPW)
    pad = ((0, 0), (0, PAD))
    src3 = jnp.pad(src, pad).reshape(NW, NCHUNK, CH)
    dstg = jnp.pad(dst, pad).reshape(NW, NCHUNK, CH)
    dsts = jnp.pad(dst, pad, constant_values=N).reshape(NW, NCHUNK, CH)
    xs, xd = _gather(x, src3, dstg)
    msg = _mlp(xs, xd, W1, b1.reshape(1, D), W2.reshape(1, D),
               b2.reshape(1, 1), W4, b4.reshape(1, D))
    partials = _scatter(msg, dsts, jnp.zeros((NPAD, D), jnp.float32))
    return _combine(partials)


# speedup vs baseline: 1.1238x; 1.1238x over previous
import jax
import jax.numpy as jnp
from jax import lax
from jax.experimental import pallas as pl
from jax.experimental.pallas import tpu as pltpu
from jax.experimental.pallas import tpu_sc as plsc

N = 10000
E = 320000
D = 128

NC = 2
NS = 16
NW = NC * NS
EPW = E // NW
CH = 128
NCHUNK = 79
EPWP = NCHUNK * CH
PAD = EPWP - EPW
EP = NW * EPWP
NPAD = 10240
RPS = NPAD // NS


def _gather_body(x_hbm, src_hbm, dst_hbm, xs_hbm, xd_hbm,
                 idx_s, idx_d, bs, bd, g_sem, w_sem):
    wid = lax.axis_index("s") * NC + lax.axis_index("c")
    pltpu.sync_copy(src_hbm.at[wid], idx_s)
    pltpu.sync_copy(dst_hbm.at[wid], idx_d)

    def gather(j, p):
        return (pltpu.make_async_copy(x_hbm.at[idx_s.at[j]], bs.at[p],
                                      g_sem.at[p]),
                pltpu.make_async_copy(x_hbm.at[idx_d.at[j]], bd.at[p],
                                      g_sem.at[p]))

    def write(j, p):
        base = wid * EPWP + j * CH
        return (pltpu.make_async_copy(bs.at[p], xs_hbm.at[pl.ds(base, CH)],
                                      w_sem.at[p]),
                pltpu.make_async_copy(bd.at[p], xd_hbm.at[pl.ds(base, CH)],
                                      w_sem.at[p]))

    for d in gather(0, 0):
        d.start()

    def step(j, carry):
        p = lax.rem(j, 2)
        q = 1 - p

        @pl.when(j + 1 < NCHUNK)
        def _():
            @pl.when(j >= 1)
            def _():
                for d in write(j - 1, q):
                    d.wait()
            for d in gather(j + 1, q):
                d.start()

        for d in gather(j, p):
            d.wait()
        for d in write(j, p):
            d.start()
        return carry

    lax.fori_loop(0, NCHUNK, step, 0)
    for j in (NCHUNK - 2, NCHUNK - 1):
        for d in write(j, j % 2):
            d.wait()


_gather = pl.kernel(
    _gather_body,
    out_type=(jax.ShapeDtypeStruct((EP, D), jnp.float32),
              jax.ShapeDtypeStruct((EP, D), jnp.float32)),
    mesh=plsc.VectorSubcoreMesh(core_axis_name="c", subcore_axis_name="s"),
    scratch_types=[
        pltpu.VMEM((NCHUNK, CH), jnp.int32),
        pltpu.VMEM((NCHUNK, CH), jnp.int32),
        pltpu.VMEM((2, CH, D), jnp.float32),
        pltpu.VMEM((2, CH, D), jnp.float32),
        pltpu.SemaphoreType.DMA((2,)),
        pltpu.SemaphoreType.DMA((2,)),
    ],
)


def _mlp_body(xs_ref, xd_ref, w1_ref, b1_ref, w2r_ref, b2_ref, w4_ref,
              b4_ref, out_ref):
    xs = xs_ref[...]
    xd = xd_ref[...]
    theta = xd - xs
    h1 = jnp.dot(theta, w1_ref[...], preferred_element_type=jnp.float32)
    h1 = jnp.maximum(h1 + b1_ref[...], 0.0)
    logit = jnp.sum(h1 * w2r_ref[...], axis=1, keepdims=True) + b2_ref[0, 0]
    w = jax.nn.sigmoid(logit)
    xlin = jnp.dot(xd, w4_ref[...], preferred_element_type=jnp.float32)
    out_ref[...] = w * theta + xlin + b4_ref[...]


BE = 4096


def _mlp(xs, xd, W1, b1r, W2r, b2r, W4, b4r):
    full = lambda shape: pl.BlockSpec(shape, lambda i: (0, 0))
    return pl.pallas_call(
        _mlp_body,
        grid=(EP // BE,),
        in_specs=[
            pl.BlockSpec((BE, D), lambda i: (i, 0)),
            pl.BlockSpec((BE, D), lambda i: (i, 0)),
            full((D, D)),
            full((1, D)),
            full((1, D)),
            pl.BlockSpec(memory_space=pltpu.SMEM),
            full((D, D)),
            full((1, D)),
        ],
        out_specs=pl.BlockSpec((BE, D), lambda i: (i, 0)),
        out_shape=jax.ShapeDtypeStruct((EP, D), jnp.float32),
    )(xs, xd, W1, b1r, W2r, b2r, W4, b4r)


def _scatter_body(m_hbm, dst_hbm, z_hbm, out_hbm, idx_v, buf, acc, l_sem):
    c = lax.axis_index("c")
    s = lax.axis_index("s")
    wid = s * NC + c
    pltpu.sync_copy(z_hbm.at[pl.ds(s * RPS, RPS)], acc.at[pl.ds(s * RPS, RPS)])
    pltpu.sync_copy(dst_hbm.at[wid], idx_v)
    plsc.subcore_barrier()

    def load(j, p):
        base = wid * EPWP + j * CH
        return pltpu.make_async_copy(m_hbm.at[pl.ds(base, CH)], buf.at[p],
                                     l_sem.at[p])

    load(0, 0).start()

    def step(j, carry):
        p = lax.rem(j, 2)

        @pl.when(j + 1 < NCHUNK)
        def _():
            load(j + 1, 1 - p).start()

        load(j, p).wait()
        pltpu.sync_copy(buf.at[p], acc.at[idx_v.at[j]], add=True)
        return carry

    lax.fori_loop(0, NCHUNK, step, 0)
    plsc.subcore_barrier()
    pltpu.sync_copy(acc.at[pl.ds(s * RPS, RPS)],
                    out_hbm.at[c, pl.ds(s * RPS, RPS)])


_scatter = pl.kernel(
    _scatter_body,
    out_type=jax.ShapeDtypeStruct((NC, NPAD, D), jnp.float32),
    mesh=plsc.VectorSubcoreMesh(core_axis_name="c", subcore_axis_name="s"),
    scratch_types=[
        pltpu.VMEM((NCHUNK, CH), jnp.int32),
        pltpu.VMEM((2, CH, D), jnp.float32),
        pltpu.VMEM_SHARED((NPAD, D), jnp.float32),
        pltpu.SemaphoreType.DMA((2,)),
    ],
)


def _combine_body(p_ref, out_ref):
    out_ref[...] = p_ref[0] + p_ref[1]


BN = 1000


def _combine(partials):
    return pl.pallas_call(
        _combine_body,
        grid=(N // BN,),
        in_specs=[pl.BlockSpec((NC, BN, D), lambda i: (0, i, 0))],
        out_specs=pl.BlockSpec((BN, D), lambda i: (i, 0)),
        out_shape=jax.ShapeDtypeStruct((N, D), jnp.float32),
    )(partials)


def kernel(x, edge_index, W1, b1, W2, b2, W4, b4):
    src = edge_index[0].reshape(NW, EPW)
    dst = edge_index[1].reshape(NW, EPW)
    pad = ((0, 0), (0, PAD))
    src3 = jnp.pad(src, pad).reshape(NW, NCHUNK, CH)
    dstg = jnp.pad(dst, pad).reshape(NW, NCHUNK, CH)
    dsts = jnp.pad(dst, pad, constant_values=N).reshape(NW, NCHUNK, CH)
    xs, xd = _gather(x, src3, dstg)
    msg = _mlp(xs, xd, W1, b1.reshape(1, D), W2.reshape(1, D),
               b2.reshape(1, 1), W4, b4.reshape(1, D))
    partials = _scatter(msg, dsts, jnp.zeros((NPAD, D), jnp.float32))
    return _combine(partials)


# speedup vs baseline: 1.1430x; 1.0171x over previous
import jax
import jax.numpy as jnp
from jax import lax
from jax.experimental import pallas as pl
from jax.experimental.pallas import tpu as pltpu
from jax.experimental.pallas import tpu_sc as plsc

N = 10000
E = 320000
D = 128

NC = 2
NS = 16
NW = NC * NS
EPW = E // NW
CH = 128
NCHUNK = 79
EPWP = NCHUNK * CH
PAD = EPWP - EPW
EP = NW * EPWP
NPAD = 10240
RPS = NPAD // NS


def _gather_body(x_hbm, src_hbm, dst_hbm, xs_hbm, xd_hbm,
                 idx_s, idx_d, bs, bd, g_sem, w_sem):
    wid = lax.axis_index("s") * NC + lax.axis_index("c")
    pltpu.sync_copy(src_hbm.at[wid], idx_s)
    pltpu.sync_copy(dst_hbm.at[wid], idx_d)

    def gather(j, p):
        return (pltpu.make_async_copy(x_hbm.at[idx_s.at[j]], bs.at[p],
                                      g_sem.at[p]),
                pltpu.make_async_copy(x_hbm.at[idx_d.at[j]], bd.at[p],
                                      g_sem.at[p]))

    def write(j, p):
        base = wid * EPWP + j * CH
        return (pltpu.make_async_copy(bs.at[p], xs_hbm.at[pl.ds(base, CH)],
                                      w_sem.at[p]),
                pltpu.make_async_copy(bd.at[p], xd_hbm.at[pl.ds(base, CH)],
                                      w_sem.at[p]))

    for d in gather(0, 0):
        d.start()

    def step(j, carry):
        p = lax.rem(j, 2)
        q = 1 - p

        @pl.when(j + 1 < NCHUNK)
        def _():
            @pl.when(j >= 1)
            def _():
                for d in write(j - 1, q):
                    d.wait()
            for d in gather(j + 1, q):
                d.start()

        for d in gather(j, p):
            d.wait()
        for d in write(j, p):
            d.start()
        return carry

    lax.fori_loop(0, NCHUNK, step, 0)
    for j in (NCHUNK - 2, NCHUNK - 1):
        for d in write(j, j % 2):
            d.wait()


_gather = pl.kernel(
    _gather_body,
    out_type=(jax.ShapeDtypeStruct((EP, D), jnp.float32),
              jax.ShapeDtypeStruct((EP, D), jnp.float32)),
    mesh=plsc.VectorSubcoreMesh(core_axis_name="c", subcore_axis_name="s"),
    scratch_types=[
        pltpu.VMEM((NCHUNK, CH), jnp.int32),
        pltpu.VMEM((NCHUNK, CH), jnp.int32),
        pltpu.VMEM((2, CH, D), jnp.float32),
        pltpu.VMEM((2, CH, D), jnp.float32),
        pltpu.SemaphoreType.DMA((2,)),
        pltpu.SemaphoreType.DMA((2,)),
    ],
)


def _mlp_body(xs_ref, xd_ref, w1_ref, b1_ref, w2r_ref, b2_ref, w4_ref,
              b4_ref, out_ref):
    xs = xs_ref[...]
    xd = xd_ref[...]
    theta = xd - xs
    h1 = jnp.dot(theta, w1_ref[...], preferred_element_type=jnp.float32)
    h1 = jnp.maximum(h1 + b1_ref[...], 0.0)
    logit = jnp.sum(h1 * w2r_ref[...], axis=1, keepdims=True) + b2_ref[0, 0]
    w = jax.nn.sigmoid(logit)
    xlin = jnp.dot(xd, w4_ref[...], preferred_element_type=jnp.float32)
    out_ref[...] = w * theta + xlin + b4_ref[...]


BE = 10112


def _mlp(xs, xd, W1, b1r, W2r, b2r, W4, b4r):
    full = lambda shape: pl.BlockSpec(shape, lambda i: (0, 0))
    return pl.pallas_call(
        _mlp_body,
        grid=(EP // BE,),
        in_specs=[
            pl.BlockSpec((BE, D), lambda i: (i, 0)),
            pl.BlockSpec((BE, D), lambda i: (i, 0)),
            full((D, D)),
            full((1, D)),
            full((1, D)),
            pl.BlockSpec(memory_space=pltpu.SMEM),
            full((D, D)),
            full((1, D)),
        ],
        out_specs=pl.BlockSpec((BE, D), lambda i: (i, 0)),
        out_shape=jax.ShapeDtypeStruct((EP, D), jnp.float32),
    )(xs, xd, W1, b1r, W2r, b2r, W4, b4r)


def _scatter_body(m_hbm, dst_hbm, z_hbm, out_hbm, idx_v, buf, acc, l_sem):
    c = lax.axis_index("c")
    s = lax.axis_index("s")
    wid = s * NC + c
    pltpu.sync_copy(z_hbm.at[pl.ds(s * RPS, RPS)], acc.at[pl.ds(s * RPS, RPS)])
    pltpu.sync_copy(dst_hbm.at[wid], idx_v)
    plsc.subcore_barrier()

    def load(j, p):
        base = wid * EPWP + j * CH
        return pltpu.make_async_copy(m_hbm.at[pl.ds(base, CH)], buf.at[p],
                                     l_sem.at[p])

    load(0, 0).start()

    def step(j, carry):
        p = lax.rem(j, 2)

        @pl.when(j + 1 < NCHUNK)
        def _():
            load(j + 1, 1 - p).start()

        load(j, p).wait()
        pltpu.sync_copy(buf.at[p], acc.at[idx_v.at[j]], add=True)
        return carry

    lax.fori_loop(0, NCHUNK, step, 0)
    plsc.subcore_barrier()
    pltpu.sync_copy(acc.at[pl.ds(s * RPS, RPS)],
                    out_hbm.at[c, pl.ds(s * RPS, RPS)])


_scatter = pl.kernel(
    _scatter_body,
    out_type=jax.ShapeDtypeStruct((NC, NPAD, D), jnp.float32),
    mesh=plsc.VectorSubcoreMesh(core_axis_name="c", subcore_axis_name="s"),
    scratch_types=[
        pltpu.VMEM((NCHUNK, CH), jnp.int32),
        pltpu.VMEM((2, CH, D), jnp.float32),
        pltpu.VMEM_SHARED((NPAD, D), jnp.float32),
        pltpu.SemaphoreType.DMA((2,)),
    ],
)


def _combine_body(p_ref, out_ref):
    out_ref[...] = p_ref[0] + p_ref[1]


BN = 1000


def _combine(partials):
    return pl.pallas_call(
        _combine_body,
        grid=(N // BN,),
        in_specs=[pl.BlockSpec((NC, BN, D), lambda i: (0, i, 0))],
        out_specs=pl.BlockSpec((BN, D), lambda i: (i, 0)),
        out_shape=jax.ShapeDtypeStruct((N, D), jnp.float32),
    )(partials)


def kernel(x, edge_index, W1, b1, W2, b2, W4, b4):
    src = edge_index[0].reshape(NW, EPW)
    dst = edge_index[1].reshape(NW, EPW)
    pad = ((0, 0), (0, PAD))
    src3 = jnp.pad(src, pad).reshape(NW, NCHUNK, CH)
    dstg = jnp.pad(dst, pad).reshape(NW, NCHUNK, CH)
    dsts = jnp.pad(dst, pad, constant_values=N).reshape(NW, NCHUNK, CH)
    xs, xd = _gather(x, src3, dstg)
    msg = _mlp(xs, xd, W1, b1.reshape(1, D), W2.reshape(1, D),
               b2.reshape(1, 1), W4, b4.reshape(1, D))
    partials = _scatter(msg, dsts, jnp.zeros((NPAD, D), jnp.float32))
    return _combine(partials)


# speedup vs baseline: 1.2732x; 1.1139x over previous
import jax
import jax.numpy as jnp
from jax import lax
from jax.experimental import pallas as pl
from jax.experimental.pallas import tpu as pltpu
from jax.experimental.pallas import tpu_sc as plsc

N = 10000
E = 320000
D = 128

NC = 2
NS = 16
NW = NC * NS
EPW = E // NW
CH = 80
NCHUNK = EPW // CH
NPAD = 10240
RPS = NPAD // NS


def _gather_body(x_hbm, src_hbm, dst_hbm, xs_hbm, xd_hbm,
                 idx_s, idx_d, buf_s, buf_d, sem):
    wid = lax.axis_index("s") * NC + lax.axis_index("c")
    pltpu.sync_copy(src_hbm.at[wid], idx_s)
    pltpu.sync_copy(dst_hbm.at[wid], idx_d)

    def step(j, carry):
        pltpu.async_copy(x_hbm.at[idx_s.at[j]], buf_s, sem).wait()
        pltpu.async_copy(x_hbm.at[idx_d.at[j]], buf_d, sem).wait()
        base = wid * EPW + j * CH
        pltpu.sync_copy(buf_s, xs_hbm.at[pl.ds(base, CH)])
        pltpu.sync_copy(buf_d, xd_hbm.at[pl.ds(base, CH)])
        return carry

    lax.fori_loop(0, NCHUNK, step, 0)


_gather = pl.kernel(
    _gather_body,
    out_type=(jax.ShapeDtypeStruct((E, D), jnp.float32),
              jax.ShapeDtypeStruct((E, D), jnp.float32)),
    mesh=plsc.VectorSubcoreMesh(core_axis_name="c", subcore_axis_name="s"),
    scratch_types=[
        pltpu.VMEM((NCHUNK, CH), jnp.int32),
        pltpu.VMEM((NCHUNK, CH), jnp.int32),
        pltpu.VMEM((CH, D), jnp.float32),
        pltpu.VMEM((CH, D), jnp.float32),
        pltpu.SemaphoreType.DMA,
    ],
)


def _mlp_body(xs_ref, xd_ref, w1_ref, b1_ref, w2r_ref, b2_ref, w4_ref,
              b4_ref, out_ref):
    xs = xs_ref[...]
    xd = xd_ref[...]
    theta = xd - xs
    h1 = jnp.dot(theta, w1_ref[...], preferred_element_type=jnp.float32)
    h1 = jnp.maximum(h1 + b1_ref[...], 0.0)
    logit = jnp.sum(h1 * w2r_ref[...], axis=1, keepdims=True) + b2_ref[0, 0]
    w = jax.nn.sigmoid(logit)
    xlin = jnp.dot(xd, w4_ref[...], preferred_element_type=jnp.float32)
    out_ref[...] = w * theta + xlin + b4_ref[...]


BE = 10000


def _mlp(xs, xd, W1, b1r, W2r, b2r, W4, b4r):
    full = lambda shape: pl.BlockSpec(shape, lambda i: (0, 0))
    return pl.pallas_call(
        _mlp_body,
        grid=(E // BE,),
        in_specs=[
            pl.BlockSpec((BE, D), lambda i: (i, 0)),
            pl.BlockSpec((BE, D), lambda i: (i, 0)),
            full((D, D)),
            full((1, D)),
            full((1, D)),
            pl.BlockSpec(memory_space=pltpu.SMEM),
            full((D, D)),
            full((1, D)),
        ],
        out_specs=pl.BlockSpec((BE, D), lambda i: (i, 0)),
        out_shape=jax.ShapeDtypeStruct((E, D), jnp.float32),
    )(xs, xd, W1, b1r, W2r, b2r, W4, b4r)


def _scatter_body(m_hbm, dst_hbm, z_hbm, out_hbm, idx_v, buf, acc, l_sem):
    c = lax.axis_index("c")
    s = lax.axis_index("s")
    wid = s * NC + c
    pltpu.sync_copy(z_hbm.at[pl.ds(s * RPS, RPS)], acc.at[pl.ds(s * RPS, RPS)])
    pltpu.sync_copy(dst_hbm.at[wid], idx_v)
    plsc.subcore_barrier()

    def load(j, p):
        base = wid * EPW + j * CH
        return pltpu.make_async_copy(m_hbm.at[pl.ds(base, CH)], buf.at[p],
                                     l_sem.at[p])

    load(0, 0).start()

    def step(j, carry):
        p = lax.rem(j, 2)

        @pl.when(j + 1 < NCHUNK)
        def _():
            load(j + 1, 1 - p).start()

        load(j, p).wait()
        pltpu.sync_copy(buf.at[p], acc.at[idx_v.at[j]], add=True)
        return carry

    lax.fori_loop(0, NCHUNK, step, 0)
    plsc.subcore_barrier()
    pltpu.sync_copy(acc.at[pl.ds(s * RPS, RPS)],
                    out_hbm.at[c, pl.ds(s * RPS, RPS)])


_scatter = pl.kernel(
    _scatter_body,
    out_type=jax.ShapeDtypeStruct((NC, NPAD, D), jnp.float32),
    mesh=plsc.VectorSubcoreMesh(core_axis_name="c", subcore_axis_name="s"),
    scratch_types=[
        pltpu.VMEM((NCHUNK, CH), jnp.int32),
        pltpu.VMEM((2, CH, D), jnp.float32),
        pltpu.VMEM_SHARED((NPAD, D), jnp.float32),
        pltpu.SemaphoreType.DMA((2,)),
    ],
)


def _combine_body(p_ref, out_ref):
    out_ref[...] = p_ref[0] + p_ref[1]


BN = 1000


def _combine(partials):
    return pl.pallas_call(
        _combine_body,
        grid=(N // BN,),
        in_specs=[pl.BlockSpec((NC, BN, D), lambda i: (0, i, 0))],
        out_specs=pl.BlockSpec((BN, D), lambda i: (i, 0)),
        out_shape=jax.ShapeDtypeStruct((N, D), jnp.float32),
    )(partials)


def kernel(x, edge_index, W1, b1, W2, b2, W4, b4):
    src3 = edge_index[0].reshape(NW, NCHUNK, CH)
    dst3 = edge_index[1].reshape(NW, NCHUNK, CH)
    xs, xd = _gather(x, src3, dst3)
    msg = _mlp(xs, xd, W1, b1.reshape(1, D), W2.reshape(1, D),
               b2.reshape(1, 1), W4, b4.reshape(1, D))
    partials = _scatter(msg, dst3, jnp.zeros((NPAD, D), jnp.float32))
    return _combine(partials)
